# baseline (speedup 1.0000x reference)
import functools

import jax
import jax.numpy as jnp
from jax import lax
from jax.experimental import pallas as pl
from jax.experimental.pallas import tpu as pltpu
from jax.experimental.pallas import tpu_sc as plsc

_D = 2048
_N = 32768
_NC = 2
_NS = 16
_NW = _NC * _NS
_ROWS = _N // _NW
_C = 8
_NCH = _ROWS // _C
_L = 16
_VPR = _D // _L
_CHUNK = _C * _D

_mesh = plsc.VectorSubcoreMesh(core_axis_name="c", subcore_axis_name="s")


@functools.partial(
    pl.kernel,
    mesh=_mesh,
    out_type=[
        jax.ShapeDtypeStruct((_N * _D,), jnp.float32),
        jax.ShapeDtypeStruct((_N,), jnp.float32),
    ],
    scratch_types=[
        pltpu.VMEM((_CHUNK,), jnp.float32),
        pltpu.VMEM((_CHUNK,), jnp.float32),
        pltpu.VMEM((_ROWS,), jnp.float32),
    ],
)
def _reverse_rows(x_hbm, y_hbm, ld_hbm, in_v, out_v, z_v):
    wid = lax.axis_index("s") * _NC + lax.axis_index("c")
    row0 = wid * _ROWS
    base0 = row0 * _D

    zero16 = jnp.zeros((_L,), jnp.float32)

    def _z(i, c):
        z_v[pl.ds(i * _L, _L)] = zero16
        return c

    lax.fori_loop(0, _ROWS // _L, _z, 0)
    pltpu.sync_copy(z_v, ld_hbm.at[pl.ds(row0, _ROWS)])

    def _chunk(ci, c):
        base = base0 + ci * _CHUNK
        pltpu.sync_copy(x_hbm.at[pl.ds(base, _CHUNK)], in_v)

        def _row(r, c2):
            rb = r * _D

            def _vec(j, c3):
                src = rb + (_VPR - 1 - j) * _L
                dst = rb + j * _L
                out_v[pl.ds(dst, _L)] = lax.rev(in_v[pl.ds(src, _L)], (0,))
                return c3

            return lax.fori_loop(0, _VPR, _vec, c2)

        lax.fori_loop(0, _C, _row, c)
        pltpu.sync_copy(out_v, y_hbm.at[pl.ds(base, _CHUNK)])
        return c

    lax.fori_loop(0, _NCH, _chunk, 0)


def kernel(x, perm):
    del perm
    y_flat, logdet = _reverse_rows(x.reshape(_N * _D))
    return (y_flat.reshape(_N, _D), logdet)


# speedup vs baseline: 2.0246x; 2.0246x over previous
import functools

import jax
import jax.numpy as jnp
from jax import lax
from jax.experimental import pallas as pl
from jax.experimental.pallas import tpu as pltpu
from jax.experimental.pallas import tpu_sc as plsc

_D = 2048
_N = 32768
_NC = 2
_NS = 16
_NW = _NC * _NS
_ROWS = _N // _NW
_C = 8
_NCH = _ROWS // _C
_L = 16
_VPR = _D // _L
_CHUNK = _C * _D

_mesh = plsc.VectorSubcoreMesh(core_axis_name="c", subcore_axis_name="s")


@functools.partial(
    pl.kernel,
    mesh=_mesh,
    out_type=[
        jax.ShapeDtypeStruct((_N * _D,), jnp.float32),
        jax.ShapeDtypeStruct((_N,), jnp.float32),
    ],
    scratch_types=[
        pltpu.VMEM((_CHUNK,), jnp.float32),
        pltpu.VMEM((_CHUNK,), jnp.float32),
        pltpu.VMEM((_CHUNK,), jnp.float32),
        pltpu.VMEM((_CHUNK,), jnp.float32),
        pltpu.VMEM((_ROWS,), jnp.float32),
        pltpu.SemaphoreType.DMA,
        pltpu.SemaphoreType.DMA,
        pltpu.SemaphoreType.DMA,
        pltpu.SemaphoreType.DMA,
    ],
)
def _reverse_rows(x_hbm, y_hbm, ld_hbm, in_v0, in_v1, out_v0, out_v1, z_v,
                  si0, si1, so0, so1):
    wid = lax.axis_index("s") * _NC + lax.axis_index("c")
    row0 = wid * _ROWS
    base0 = row0 * _D
    in_bufs = (in_v0, in_v1)
    out_bufs = (out_v0, out_v1)
    sins = (si0, si1)
    souts = (so0, so1)

    zero16 = jnp.zeros((_L,), jnp.float32)

    @pl.loop(0, _ROWS // _L)
    def _z(i):
        z_v[pl.ds(i * _L, _L)] = zero16

    pltpu.sync_copy(z_v, ld_hbm.at[pl.ds(row0, _ROWS)])

    pltpu.async_copy(x_hbm.at[pl.ds(base0, _CHUNK)], in_v0, si0)
    pltpu.async_copy(x_hbm.at[pl.ds(base0 + _CHUNK, _CHUNK)], in_v1, si1)

    @pl.loop(0, _NCH, step=2)
    def _g(g):
        for b in range(2):
            ci = g + b
            ib, ob, si, so = in_bufs[b], out_bufs[b], sins[b], souts[b]
            base = base0 + ci * _CHUNK
            pltpu.make_async_copy(x_hbm.at[pl.ds(base, _CHUNK)], ib, si).wait()

            @pl.when(ci >= 2)
            def _wait_out():
                pltpu.make_async_copy(
                    ob, y_hbm.at[pl.ds(base - 2 * _CHUNK, _CHUNK)], so).wait()

            @plsc.parallel_loop(0, _C * _VPR, unroll=8)
            def _vec(t):
                src = (t + (_VPR - 1) - 2 * (t & (_VPR - 1))) * _L
                ob[pl.ds(t * _L, _L)] = lax.rev(ib[pl.ds(src, _L)], (0,))

            pltpu.async_copy(ob, y_hbm.at[pl.ds(base, _CHUNK)], so)

            @pl.when(ci + 2 < _NCH)
            def _next_in():
                pltpu.async_copy(
                    x_hbm.at[pl.ds(base + 2 * _CHUNK, _CHUNK)], ib, si)

    for b in range(2):
        base = base0 + (_NCH - 2 + b) * _CHUNK
        pltpu.make_async_copy(
            out_bufs[b], y_hbm.at[pl.ds(base, _CHUNK)], souts[b]).wait()


def kernel(x, perm):
    del perm
    y_flat, logdet = _reverse_rows(x.reshape(_N * _D))
    return (y_flat.reshape(_N, _D), logdet)


# speedup vs baseline: 6.4912x; 3.2062x over previous
import functools

import jax
import jax.numpy as jnp
from jax import lax
from jax.experimental import pallas as pl
from jax.experimental.pallas import tpu as pltpu
from jax.experimental.pallas import tpu_sc as plsc

_D = 2048
_N = 32768
_NC = 2
_NS = 16
_NW = _NC * _NS
_ROWS = _N // _NW
_C = 8
_NCH = _ROWS // _C
_L = 16
_VPR = _D // _L

_mesh = plsc.VectorSubcoreMesh(core_axis_name="c", subcore_axis_name="s")


@functools.partial(
    pl.kernel,
    mesh=_mesh,
    out_type=[
        jax.ShapeDtypeStruct((_N, _D), jnp.float32),
        jax.ShapeDtypeStruct((_N,), jnp.float32),
    ],
    scratch_types=[
        pltpu.VMEM((_C, _D), jnp.float32),
        pltpu.VMEM((_C, _D), jnp.float32),
        pltpu.VMEM((_C, _D), jnp.float32),
        pltpu.VMEM((_C, _D), jnp.float32),
        pltpu.VMEM((_ROWS,), jnp.float32),
        pltpu.SemaphoreType.DMA,
        pltpu.SemaphoreType.DMA,
        pltpu.SemaphoreType.DMA,
        pltpu.SemaphoreType.DMA,
    ],
)
def _reverse_rows(x_hbm, y_hbm, ld_hbm, in_v0, in_v1, out_v0, out_v1, z_v,
                  si0, si1, so0, so1):
    wid = lax.axis_index("s") * _NC + lax.axis_index("c")
    row0 = wid * _ROWS
    in_bufs = (in_v0, in_v1)
    out_bufs = (out_v0, out_v1)
    sins = (si0, si1)
    souts = (so0, so1)

    zero16 = jnp.zeros((_L,), jnp.float32)

    @pl.loop(0, _ROWS // _L)
    def _z(i):
        z_v[pl.ds(i * _L, _L)] = zero16

    pltpu.sync_copy(z_v, ld_hbm.at[pl.ds(row0, _ROWS)])

    pltpu.async_copy(x_hbm.at[pl.ds(row0, _C)], in_v0, si0)
    pltpu.async_copy(x_hbm.at[pl.ds(row0 + _C, _C)], in_v1, si1)

    @pl.loop(0, _NCH, step=2)
    def _g(g):
        for b in range(2):
            ci = g + b
            ib, ob, si, so = in_bufs[b], out_bufs[b], sins[b], souts[b]
            row = row0 + ci * _C
            pltpu.make_async_copy(x_hbm.at[pl.ds(row, _C)], ib, si).wait()

            @pl.when(ci >= 2)
            def _wait_out():
                pltpu.make_async_copy(
                    ob, y_hbm.at[pl.ds(row - 2 * _C, _C)], so).wait()

            @plsc.parallel_loop(0, _C * _VPR, unroll=8)
            def _vec(t):
                r = t >> 7
                j = t & (_VPR - 1)
                src = (_VPR - 1 - j) * _L
                ob[r, pl.ds(j * _L, _L)] = lax.rev(ib[r, pl.ds(src, _L)], (0,))

            pltpu.async_copy(ob, y_hbm.at[pl.ds(row, _C)], so)

            @pl.when(ci + 2 < _NCH)
            def _next_in():
                pltpu.async_copy(x_hbm.at[pl.ds(row + 2 * _C, _C)], ib, si)

    for b in range(2):
        row = row0 + (_NCH - 2 + b) * _C
        pltpu.make_async_copy(
            out_bufs[b], y_hbm.at[pl.ds(row, _C)], souts[b]).wait()


def kernel(x, perm):
    del perm
    y, logdet = _reverse_rows(x)
    return (y, logdet)


# speedup vs baseline: 6.4973x; 1.0009x over previous
import functools

import jax
import jax.numpy as jnp
from jax import lax
from jax.experimental import pallas as pl
from jax.experimental.pallas import tpu as pltpu
from jax.experimental.pallas import tpu_sc as plsc

_D = 2048
_N = 32768
_NC = 2
_NS = 16
_NW = _NC * _NS
_ROWS = _N // _NW
_C = 8
_NCH = _ROWS // _C
_L = 16
_VPR = _D // _L

_mesh = plsc.VectorSubcoreMesh(core_axis_name="c", subcore_axis_name="s")


@functools.partial(
    pl.kernel,
    mesh=_mesh,
    out_type=[
        jax.ShapeDtypeStruct((_N, _D), jnp.float32),
        jax.ShapeDtypeStruct((_N,), jnp.float32),
    ],
    scratch_types=[
        pltpu.VMEM((_C, _D), jnp.float32),
        pltpu.VMEM((_C, _D), jnp.float32),
        pltpu.VMEM((_C, _D), jnp.float32),
        pltpu.VMEM((_C, _D), jnp.float32),
        pltpu.VMEM((_ROWS,), jnp.float32),
        pltpu.SemaphoreType.DMA,
        pltpu.SemaphoreType.DMA,
        pltpu.SemaphoreType.DMA,
        pltpu.SemaphoreType.DMA,
    ],
)
def _reverse_rows(x_hbm, y_hbm, ld_hbm, in_v0, in_v1, out_v0, out_v1, z_v,
                  si0, si1, so0, so1):
    wid = lax.axis_index("s") * _NC + lax.axis_index("c")
    row0 = wid * _ROWS
    in_bufs = (in_v0, in_v1)
    out_bufs = (out_v0, out_v1)
    sins = (si0, si1)
    souts = (so0, so1)

    zero16 = jnp.zeros((_L,), jnp.float32)

    @pl.loop(0, _ROWS // _L)
    def _z(i):
        z_v[pl.ds(i * _L, _L)] = zero16

    pltpu.sync_copy(z_v, ld_hbm.at[pl.ds(row0, _ROWS)])

    pltpu.async_copy(x_hbm.at[pl.ds(row0, _C)], in_v0, si0)
    pltpu.async_copy(x_hbm.at[pl.ds(row0 + _C, _C)], in_v1, si1)

    @pl.loop(0, _NCH, step=2)
    def _g(g):
        for b in range(2):
            ci = g + b
            ib, ob, si, so = in_bufs[b], out_bufs[b], sins[b], souts[b]
            row = row0 + ci * _C
            pltpu.make_async_copy(x_hbm.at[pl.ds(row, _C)], ib, si).wait()

            @pl.when(ci >= 2)
            def _wait_out():
                pltpu.make_async_copy(
                    ob, y_hbm.at[pl.ds(row - 2 * _C, _C)], so).wait()

            @plsc.parallel_loop(0, _C * _VPR, unroll=16)
            def _vec(t):
                r = t >> 7
                j = t & (_VPR - 1)
                src = (_VPR - 1 - j) * _L
                ob[r, pl.ds(j * _L, _L)] = lax.rev(ib[r, pl.ds(src, _L)], (0,))

            pltpu.async_copy(ob, y_hbm.at[pl.ds(row, _C)], so)

            @pl.when(ci + 2 < _NCH)
            def _next_in():
                pltpu.async_copy(x_hbm.at[pl.ds(row + 2 * _C, _C)], ib, si)

    for b in range(2):
        row = row0 + (_NCH - 2 + b) * _C
        pltpu.make_async_copy(
            out_bufs[b], y_hbm.at[pl.ds(row, _C)], souts[b]).wait()


def kernel(x, perm):
    del perm
    y, logdet = _reverse_rows(x)
    return (y, logdet)
